# Initial kernel scaffold; baseline (speedup 1.0000x reference)
import functools

import jax
import jax.numpy as jnp
from jax import lax
from jax.experimental import pallas as pl
from jax.experimental.pallas import tpu as pltpu
from jax.experimental.pallas import tpu_sc as plsc

NC, NS, L = 2, 16, 16
NW = NC * NS

IVLEN = 128
CR = 3
C = CR * IVLEN


def _build_sc_call(n_out, hid):
    assert n_out % (NW * IVLEN) == 0
    rows_per_w = n_out // NW
    assert rows_per_w % C == 0
    g_chunks = rows_per_w // C
    irows_per_w = rows_per_w // IVLEN

    mesh = plsc.VectorSubcoreMesh(
        core_axis_name="c", subcore_axis_name="s",
        num_cores=NC, num_subcores=NS)

    @functools.partial(
        pl.kernel,
        out_type=jax.ShapeDtypeStruct((n_out, hid), jnp.float32),
        mesh=mesh,
        scratch_types=[
            pltpu.VMEM((CR, IVLEN), jnp.int32),
            pltpu.VMEM((CR, IVLEN), jnp.int32),
            pltpu.VMEM((C, hid), jnp.float32),
            pltpu.VMEM((C, hid), jnp.float32),
            pltpu.SemaphoreType.DMA,
        ],
    )
    def emb(idx0_hbm, idx1_hbm, tab_hbm, out_hbm, i0_v, i1_v, r0_v, r1_v, sem):
        wid = lax.axis_index("s") * NC + lax.axis_index("c")
        wrow = wid * irows_per_w
        wbase = wid * rows_per_w

        def row_body(j, carry):
            for k2 in range(hid // L):
                sl = pl.ds(k2 * L, L)
                r0_v[j, sl] = (r0_v[j, sl] + r1_v[j, sl]) * 0.5
            return carry

        def chunk_body(g, carry):
            row_off = wrow + g * CR
            pltpu.sync_copy(idx0_hbm.at[pl.ds(row_off, CR)], i0_v)
            pltpu.sync_copy(idx1_hbm.at[pl.ds(row_off, CR)], i1_v)
            cps = []
            for k in range(CR):
                dst = pl.ds(k * IVLEN, IVLEN)
                cps.append(pltpu.async_copy(
                    tab_hbm.at[i0_v.at[k]], r0_v.at[dst], sem))
                cps.append(pltpu.async_copy(
                    tab_hbm.at[i1_v.at[k]], r1_v.at[dst], sem))
            for cp in cps:
                cp.wait()
            lax.fori_loop(0, C, row_body, 0, unroll=False)
            pltpu.sync_copy(r0_v, out_hbm.at[pl.ds(wbase + g * C, C)])
            return carry

        lax.fori_loop(0, g_chunks, chunk_body, 0, unroll=False)

    return emb


def kernel(time, time_embed_weight):
    b, s, td = time.shape
    vocab, hid = time_embed_weight.shape
    assert td == 2 and hid % L == 0
    n_out = b * s
    idx = time.reshape(n_out, td).astype(jnp.int32)
    idx0 = idx[:, 0].reshape(n_out // IVLEN, IVLEN)
    idx1 = idx[:, 1].reshape(n_out // IVLEN, IVLEN)
    out = _build_sc_call(n_out, hid)(idx0, idx1, time_embed_weight)
    return out.reshape(b, s, hid)


# speedup vs baseline: 8.9887x; 8.9887x over previous
import functools

import jax
import jax.numpy as jnp
from jax import lax
from jax.experimental import pallas as pl
from jax.experimental.pallas import tpu as pltpu
from jax.experimental.pallas import tpu_sc as plsc

NC, NS, L = 2, 16, 16
NW = NC * NS

IVLEN = 128
CR = 3
C = CR * IVLEN


def _build_sc_call(n_out, hid):
    assert n_out % (NW * IVLEN) == 0
    rows_per_w = n_out // NW
    assert rows_per_w % C == 0
    g_chunks = rows_per_w // C

    mesh = plsc.VectorSubcoreMesh(
        core_axis_name="c", subcore_axis_name="s",
        num_cores=NC, num_subcores=NS)

    @functools.partial(
        pl.kernel,
        out_type=jax.ShapeDtypeStruct((n_out, hid), jnp.float32),
        mesh=mesh,
        scratch_types=[
            pltpu.VMEM((C,), jnp.int32),
            pltpu.VMEM((C,), jnp.int32),
            pltpu.VMEM((C, hid), jnp.float32),
            pltpu.VMEM((C, hid), jnp.float32),
            pltpu.SemaphoreType.DMA,
        ],
        compiler_params=pltpu.CompilerParams(use_tc_tiling_on_sc=False),
    )
    def emb(idx0_hbm, idx1_hbm, tab_hbm, out_hbm, i0_v, i1_v, r0_v, r1_v, sem):
        wid = lax.axis_index("s") * NC + lax.axis_index("c")
        wbase = wid * rows_per_w

        def row_body(j, carry):
            for k2 in range(hid // L):
                sl = pl.ds(k2 * L, L)
                r0_v[j, sl] = (r0_v[j, sl] + r1_v[j, sl]) * 0.5
            return carry

        def chunk_body(g, carry):
            off = wbase + g * C
            pltpu.sync_copy(idx0_hbm.at[pl.ds(off, C)], i0_v)
            pltpu.sync_copy(idx1_hbm.at[pl.ds(off, C)], i1_v)
            cps = []
            for k in range(CR):
                dst = pl.ds(k * IVLEN, IVLEN)
                src = pl.ds(k * IVLEN, IVLEN)
                cps.append(pltpu.async_copy(
                    tab_hbm.at[i0_v.at[src]], r0_v.at[dst], sem))
                cps.append(pltpu.async_copy(
                    tab_hbm.at[i1_v.at[src]], r1_v.at[dst], sem))
            for cp in cps:
                cp.wait()
            lax.fori_loop(0, C, row_body, 0, unroll=False)
            pltpu.sync_copy(r0_v, out_hbm.at[pl.ds(wbase + g * C, C)])
            return carry

        lax.fori_loop(0, g_chunks, chunk_body, 0, unroll=False)

    return emb


def kernel(time, time_embed_weight):
    b, s, td = time.shape
    vocab, hid = time_embed_weight.shape
    assert td == 2 and hid % L == 0
    n_out = b * s
    idx = time.reshape(n_out, td).astype(jnp.int32)
    idx0 = idx[:, 0]
    idx1 = idx[:, 1]
    out = _build_sc_call(n_out, hid)(idx0, idx1, time_embed_weight)
    return out.reshape(b, s, hid)


# speedup vs baseline: 9.2964x; 1.0342x over previous
import functools

import jax
import jax.numpy as jnp
from jax import lax
from jax.experimental import pallas as pl
from jax.experimental.pallas import tpu as pltpu
from jax.experimental.pallas import tpu_sc as plsc

NC, NS, L = 2, 16, 16
NW = NC * NS

IVLEN = 128
CR = 3
C = CR * IVLEN
NBUF = 2


def _build_sc_call(n_out, hid):
    assert n_out % (NW * C) == 0
    rows_per_w = n_out // NW
    g_chunks = rows_per_w // C
    assert g_chunks >= 4 and (g_chunks - 3) % 2 == 0

    mesh = plsc.VectorSubcoreMesh(
        core_axis_name="c", subcore_axis_name="s",
        num_cores=NC, num_subcores=NS)

    @functools.partial(
        pl.kernel,
        out_type=jax.ShapeDtypeStruct((n_out, hid), jnp.float32),
        mesh=mesh,
        scratch_types=[
            pltpu.VMEM((NBUF, C), jnp.int32),
            pltpu.VMEM((NBUF, C), jnp.int32),
            pltpu.VMEM((NBUF, C, hid), jnp.float32),
            pltpu.VMEM((NBUF, C, hid), jnp.float32),
            pltpu.SemaphoreType.DMA,
            pltpu.SemaphoreType.DMA,
            pltpu.SemaphoreType.DMA,
            pltpu.SemaphoreType.DMA,
            pltpu.SemaphoreType.DMA,
            pltpu.SemaphoreType.DMA,
        ],
        compiler_params=pltpu.CompilerParams(use_tc_tiling_on_sc=False),
    )
    def emb(idx0_hbm, idx1_hbm, tab_hbm, out_hbm,
            i0_v, i1_v, r0_v, r1_v,
            semi0, semi1, semg0, semg1, semw0, semw1):
        semi = (semi0, semi1)
        semg = (semg0, semg1)
        semw = (semw0, semw1)
        wid = lax.axis_index("s") * NC + lax.axis_index("c")
        wbase = wid * rows_per_w

        def prefetch_idx(cg, b):
            off = wbase + cg * C
            pltpu.async_copy(idx0_hbm.at[pl.ds(off, C)], i0_v.at[b], semi[b])
            pltpu.async_copy(idx1_hbm.at[pl.ds(off, C)], i1_v.at[b], semi[b])

        def wait_idx(cg, b):
            off = wbase + cg * C
            pltpu.make_async_copy(
                idx0_hbm.at[pl.ds(off, C)], i0_v.at[b], semi[b]).wait()
            pltpu.make_async_copy(
                idx1_hbm.at[pl.ds(off, C)], i1_v.at[b], semi[b]).wait()

        def fire_gathers(b):
            for k in range(CR):
                sl = pl.ds(k * IVLEN, IVLEN)
                pltpu.async_copy(
                    tab_hbm.at[i0_v.at[b].at[sl]], r0_v.at[b].at[sl], semg[b])
                pltpu.async_copy(
                    tab_hbm.at[i1_v.at[b].at[sl]], r1_v.at[b].at[sl], semg[b])

        def wait_gathers(b):
            for k in range(CR):
                sl = pl.ds(k * IVLEN, IVLEN)
                pltpu.make_async_copy(
                    tab_hbm.at[i0_v.at[b].at[sl]], r0_v.at[b].at[sl],
                    semg[b]).wait()
                pltpu.make_async_copy(
                    tab_hbm.at[i1_v.at[b].at[sl]], r1_v.at[b].at[sl],
                    semg[b]).wait()

        def compute(b):
            def row_body(j, carry):
                for k2 in range(hid // L):
                    sl = pl.ds(k2 * L, L)
                    r0_v[b, j, sl] = (r0_v[b, j, sl] + r1_v[b, j, sl]) * 0.5
                return carry
            lax.fori_loop(0, C, row_body, 0, unroll=2)

        def start_wb(cg, b):
            off = wbase + cg * C
            pltpu.async_copy(r0_v.at[b], out_hbm.at[pl.ds(off, C)], semw[b])

        def drain_wb(cg, b):
            off = wbase + cg * C
            pltpu.make_async_copy(
                r0_v.at[b], out_hbm.at[pl.ds(off, C)], semw[b]).wait()

        prefetch_idx(0, 0)
        prefetch_idx(1, 1)
        wait_idx(0, 0)
        fire_gathers(0)

        wait_idx(1, 1)
        fire_gathers(1)
        wait_gathers(0)
        prefetch_idx(2, 0)
        compute(0)
        start_wb(0, 0)

        def pair_body(g, carry):
            c1 = 1 + 2 * g
            drain_wb(c1 - 1, 0)
            wait_idx(c1 + 1, 0)
            fire_gathers(0)
            wait_gathers(1)
            prefetch_idx(c1 + 2, 1)
            compute(1)
            start_wb(c1, 1)

            c2 = c1 + 1
            drain_wb(c2 - 1, 1)
            wait_idx(c2 + 1, 1)
            fire_gathers(1)
            wait_gathers(0)
            prefetch_idx(c2 + 2, 0)
            compute(0)
            start_wb(c2, 0)
            return carry

        lax.fori_loop(0, (g_chunks - 3) // 2, pair_body, 0, unroll=False)

        cl = g_chunks - 2
        drain_wb(cl - 1, 0)
        wait_idx(cl + 1, 0)
        fire_gathers(0)
        wait_gathers(1)
        compute(1)
        start_wb(cl, 1)

        wait_gathers(0)
        compute(0)
        start_wb(g_chunks - 1, 0)

        drain_wb(cl, 1)
        drain_wb(g_chunks - 1, 0)

    return emb


def kernel(time, time_embed_weight):
    b, s, td = time.shape
    vocab, hid = time_embed_weight.shape
    assert td == 2 and hid % L == 0
    n_out = b * s
    idx = time.reshape(n_out, td).astype(jnp.int32)
    idx0 = idx[:, 0]
    idx1 = idx[:, 1]
    out = _build_sc_call(n_out, hid)(idx0, idx1, time_embed_weight)
    return out.reshape(b, s, hid)
